# baseline (speedup 1.0000x reference)
import functools

import jax
import jax.numpy as jnp
from jax import lax
from jax.experimental import pallas as pl
from jax.experimental.pallas import tpu as pltpu
from jax.experimental.pallas import tpu_sc as plsc

NUM_EMB = 1024
DIM = 64
CCOST = 0.25
NB, NT = 16, 576
ROWS = NB * NT
PADW = 128

_info = plsc.get_sparse_core_info()
_NC, _NS = _info.num_cores, _info.num_subcores
NW = _NC * _NS
BPW = ROWS // NW


def _vq_body(zt_ref, cbt_ref, idx_ref, loss_ref, cbp_ref):
    i = pl.program_id(0)
    cbt = cbt_ref[...]

    @pl.when(i == 0)
    def _init():
        loss_ref[0, 0] = 0.0
        cbp_ref[:, :DIM] = cbt.T
        cbp_ref[:, DIM:] = jnp.zeros((NUM_EMB, PADW - DIM), jnp.float32)

    zt = zt_ref[...].reshape(DIM, NT)
    c2 = jnp.sum(cbt * cbt, axis=0)
    z2 = jnp.sum(zt * zt, axis=0)
    mt = jax.lax.dot_general(cbt, zt, (((0,), (0,)), ((), ())),
                             preferred_element_type=jnp.float32)
    scores = (z2[None, :] - 2.0 * mt) + c2[:, None]
    val = scores
    ind = lax.broadcasted_iota(jnp.int32, scores.shape, 0)
    h = NUM_EMB
    while h > 1:
        h //= 2
        lo_v, hi_v = val[:h], val[h:]
        lo_i, hi_i = ind[:h], ind[h:]
        take_hi = (hi_v < lo_v) | ((hi_v == lo_v) & (hi_i < lo_i))
        val = jnp.where(take_hi, hi_v, lo_v)
        ind = jnp.where(take_hi, hi_i, lo_i)
    dmin = val[0]
    idx_ref[0, 0, :] = ind[0]
    loss_ref[0, 0] += jnp.sum(dmin)


_sc_mesh = plsc.VectorSubcoreMesh(core_axis_name="c", subcore_axis_name="s")


@functools.partial(
    pl.kernel, mesh=_sc_mesh,
    out_type=jax.ShapeDtypeStruct((ROWS, PADW), jnp.float32),
    scratch_types=[
        pltpu.VMEM((BPW,), jnp.int32),
        pltpu.VMEM((BPW, PADW), jnp.float32),
        pltpu.SemaphoreType.DMA,
    ],
)
def _sc_gather(cb_hbm, idx_hbm, out_hbm, idx_v, rows_v, sem):
    wid = lax.axis_index("s") * _NC + lax.axis_index("c")
    base = wid * BPW
    pltpu.sync_copy(idx_hbm.at[pl.ds(base, BPW)], idx_v)
    pltpu.async_copy(cb_hbm.at[idx_v], rows_v, sem).wait()
    pltpu.sync_copy(rows_v, out_hbm.at[pl.ds(base, BPW)])


def kernel(z, codebook):
    zt = jnp.swapaxes(z, 1, 2)
    cbt = codebook.T
    idx, loss_sum, cb_pad = pl.pallas_call(
        _vq_body,
        grid=(NB,),
        in_specs=[
            pl.BlockSpec((1, DIM, NT), lambda i: (i, 0, 0)),
            pl.BlockSpec((DIM, NUM_EMB), lambda i: (0, 0)),
        ],
        out_specs=[
            pl.BlockSpec((1, 1, NT), lambda i: (i, 0, 0)),
            pl.BlockSpec(memory_space=pltpu.SMEM,
                         block_shape=(1, 1), index_map=lambda i: (0, 0)),
            pl.BlockSpec((NUM_EMB, PADW), lambda i: (0, 0)),
        ],
        out_shape=[
            jax.ShapeDtypeStruct((NB, 1, NT), jnp.int32),
            jax.ShapeDtypeStruct((1, 1), jnp.float32),
            jax.ShapeDtypeStruct((NUM_EMB, PADW), jnp.float32),
        ],
    )(zt, cbt)
    idx_flat = idx.reshape(ROWS)
    q = _sc_gather(cb_pad, idx_flat)
    quantized = q[:, :DIM].reshape(z.shape)
    indices = idx_flat.reshape(z.shape[0], -1)
    loss = (1.0 + CCOST) * loss_sum[0, 0] / (ROWS * DIM)
    return quantized, indices, loss


# speedup vs baseline: 1.0549x; 1.0549x over previous
import functools

import jax
import jax.numpy as jnp
from jax import lax
from jax.experimental import pallas as pl
from jax.experimental.pallas import tpu as pltpu
from jax.experimental.pallas import tpu_sc as plsc

NUM_EMB = 1024
DIM = 64
CCOST = 0.25
NB, NT = 16, 576
ROWS = NB * NT
BB = 2
NBLK = NB // BB
PADW = 128

_info = plsc.get_sparse_core_info()
_NC, _NS = _info.num_cores, _info.num_subcores
NW = _NC * _NS
BPW = ROWS // NW


def _vq_body(zt_ref, cbt_ref, idx_ref, loss_ref, cbp_ref):
    i = pl.program_id(0)
    cbt = cbt_ref[...]

    @pl.when(i == 0)
    def _init():
        loss_ref[0, 0] = 0.0
        cbp_ref[:, :DIM] = cbt.T
        cbp_ref[:, DIM:] = jnp.zeros((NUM_EMB, PADW - DIM), jnp.float32)

    c2 = jnp.sum(cbt * cbt, axis=0)
    part = jnp.float32(0.0)
    for b in range(BB):
        zt = zt_ref[b]
        z2 = jnp.sum(zt * zt, axis=0)
        mt = jax.lax.dot_general(cbt, zt, (((0,), (0,)), ((), ())),
                                 preferred_element_type=jnp.float32)
        scores = (z2[None, :] - 2.0 * mt) + c2[:, None]
        dmin = jnp.min(scores, axis=0)
        riota = lax.broadcasted_iota(jnp.int32, scores.shape, 0)
        idx = jnp.min(jnp.where(scores == dmin[None, :], riota, NUM_EMB),
                      axis=0).astype(jnp.int32)
        idx_ref[0, b, :] = idx
        part = part + jnp.sum(dmin)

    loss_ref[0, 0] += part


_sc_mesh = plsc.VectorSubcoreMesh(core_axis_name="c", subcore_axis_name="s")


@functools.partial(
    pl.kernel, mesh=_sc_mesh,
    out_type=jax.ShapeDtypeStruct((ROWS, PADW), jnp.float32),
    scratch_types=[
        pltpu.VMEM((BPW,), jnp.int32),
        pltpu.VMEM((BPW, PADW), jnp.float32),
        pltpu.SemaphoreType.DMA,
    ],
)
def _sc_gather(cb_hbm, idx_hbm, out_hbm, idx_v, rows_v, sem):
    wid = lax.axis_index("s") * _NC + lax.axis_index("c")
    base = wid * BPW
    pltpu.sync_copy(idx_hbm.at[pl.ds(base, BPW)], idx_v)
    pltpu.async_copy(cb_hbm.at[idx_v], rows_v, sem).wait()
    pltpu.sync_copy(rows_v, out_hbm.at[pl.ds(base, BPW)])


def kernel(z, codebook):
    zt = jnp.swapaxes(z, 1, 2)
    cbt = codebook.T
    idx, loss_sum, cb_pad = pl.pallas_call(
        _vq_body,
        grid=(NBLK,),
        in_specs=[
            pl.BlockSpec((BB, DIM, NT), lambda i: (i, 0, 0)),
            pl.BlockSpec((DIM, NUM_EMB), lambda i: (0, 0)),
        ],
        out_specs=[
            pl.BlockSpec((1, BB, NT), lambda i: (i, 0, 0)),
            pl.BlockSpec(memory_space=pltpu.SMEM,
                         block_shape=(1, 1), index_map=lambda i: (0, 0)),
            pl.BlockSpec((NUM_EMB, PADW), lambda i: (0, 0)),
        ],
        out_shape=[
            jax.ShapeDtypeStruct((NBLK, BB, NT), jnp.int32),
            jax.ShapeDtypeStruct((1, 1), jnp.float32),
            jax.ShapeDtypeStruct((NUM_EMB, PADW), jnp.float32),
        ],
    )(zt, cbt)
    idx_flat = idx.reshape(ROWS)
    q = _sc_gather(cb_pad, idx_flat)
    quantized = q[:, :DIM].reshape(z.shape)
    indices = idx_flat.reshape(z.shape[0], -1)
    loss = (1.0 + CCOST) * loss_sum[0, 0] / (ROWS * DIM)
    return quantized, indices, loss


# speedup vs baseline: 1.0887x; 1.0320x over previous
import functools

import jax
import jax.numpy as jnp
from jax import lax
from jax.experimental import pallas as pl
from jax.experimental.pallas import tpu as pltpu
from jax.experimental.pallas import tpu_sc as plsc

NUM_EMB = 1024
DIM = 64
CCOST = 0.25
NB, NT = 16, 576
ROWS = NB * NT
BB = 4
NBLK = NB // BB
PADW = 128

_info = plsc.get_sparse_core_info()
_NC, _NS = _info.num_cores, _info.num_subcores
NW = _NC * _NS
BPW = ROWS // NW


def _vq_body(zt_ref, cbt_ref, idx_ref, loss_ref, cbp_ref):
    i = pl.program_id(0)
    cbt = cbt_ref[...]

    @pl.when(i == 0)
    def _init():
        loss_ref[0, 0] = 0.0
        cbp_ref[:, :DIM] = cbt.T
        cbp_ref[:, DIM:] = jnp.zeros((NUM_EMB, PADW - DIM), jnp.float32)

    c2 = jnp.sum(cbt * cbt, axis=0)
    part = jnp.float32(0.0)
    for b in range(BB):
        zt = zt_ref[b]
        z2 = jnp.sum(zt * zt, axis=0)
        mt = jax.lax.dot_general(cbt, zt, (((0,), (0,)), ((), ())),
                                 preferred_element_type=jnp.float32)
        scores = (z2[None, :] - 2.0 * mt) + c2[:, None]
        dmin = jnp.min(scores, axis=0)
        riota = lax.broadcasted_iota(jnp.int32, scores.shape, 0)
        idx = jnp.min(jnp.where(scores == dmin[None, :], riota, NUM_EMB),
                      axis=0).astype(jnp.int32)
        idx_ref[0, b, :] = idx
        part = part + jnp.sum(dmin)

    loss_ref[0, 0] += part


_sc_mesh = plsc.VectorSubcoreMesh(core_axis_name="c", subcore_axis_name="s")


@functools.partial(
    pl.kernel, mesh=_sc_mesh,
    out_type=jax.ShapeDtypeStruct((ROWS, PADW), jnp.float32),
    scratch_types=[
        pltpu.VMEM((BPW,), jnp.int32),
        pltpu.VMEM((BPW, PADW), jnp.float32),
        pltpu.SemaphoreType.DMA,
    ],
)
def _sc_gather(cb_hbm, idx_hbm, out_hbm, idx_v, rows_v, sem):
    wid = lax.axis_index("s") * _NC + lax.axis_index("c")
    base = wid * BPW
    pltpu.sync_copy(idx_hbm.at[pl.ds(base, BPW)], idx_v)
    pltpu.async_copy(cb_hbm.at[idx_v], rows_v, sem).wait()
    pltpu.sync_copy(rows_v, out_hbm.at[pl.ds(base, BPW)])


def kernel(z, codebook):
    zt = jnp.swapaxes(z, 1, 2)
    cbt = codebook.T
    idx, loss_sum, cb_pad = pl.pallas_call(
        _vq_body,
        grid=(NBLK,),
        in_specs=[
            pl.BlockSpec((BB, DIM, NT), lambda i: (i, 0, 0)),
            pl.BlockSpec((DIM, NUM_EMB), lambda i: (0, 0)),
        ],
        out_specs=[
            pl.BlockSpec((1, BB, NT), lambda i: (i, 0, 0)),
            pl.BlockSpec(memory_space=pltpu.SMEM,
                         block_shape=(1, 1), index_map=lambda i: (0, 0)),
            pl.BlockSpec((NUM_EMB, PADW), lambda i: (0, 0)),
        ],
        out_shape=[
            jax.ShapeDtypeStruct((NBLK, BB, NT), jnp.int32),
            jax.ShapeDtypeStruct((1, 1), jnp.float32),
            jax.ShapeDtypeStruct((NUM_EMB, PADW), jnp.float32),
        ],
    )(zt, cbt)
    idx_flat = idx.reshape(ROWS)
    q = _sc_gather(cb_pad, idx_flat)
    quantized = q[:, :DIM].reshape(z.shape)
    indices = idx_flat.reshape(z.shape[0], -1)
    loss = (1.0 + CCOST) * loss_sum[0, 0] / (ROWS * DIM)
    return quantized, indices, loss


# speedup vs baseline: 1.0893x; 1.0006x over previous
import functools

import jax
import jax.numpy as jnp
from jax import lax
from jax.experimental import pallas as pl
from jax.experimental.pallas import tpu as pltpu
from jax.experimental.pallas import tpu_sc as plsc

NUM_EMB = 1024
DIM = 64
CCOST = 0.25
NB, NT = 16, 576
ROWS = NB * NT
BB = 8
NBLK = NB // BB
PADW = 128

_info = plsc.get_sparse_core_info()
_NC, _NS = _info.num_cores, _info.num_subcores
NW = _NC * _NS
BPW = ROWS // NW


def _vq_body(zt_ref, cbt_ref, idx_ref, loss_ref, cbp_ref):
    i = pl.program_id(0)
    cbt = cbt_ref[...]

    @pl.when(i == 0)
    def _init():
        loss_ref[0, 0] = 0.0
        cbp_ref[:, :DIM] = cbt.T
        cbp_ref[:, DIM:] = jnp.zeros((NUM_EMB, PADW - DIM), jnp.float32)

    c2 = jnp.sum(cbt * cbt, axis=0)
    part = jnp.float32(0.0)
    for b in range(BB):
        zt = zt_ref[b]
        z2 = jnp.sum(zt * zt, axis=0)
        mt = jax.lax.dot_general(cbt, zt, (((0,), (0,)), ((), ())),
                                 preferred_element_type=jnp.float32)
        scores = (z2[None, :] - 2.0 * mt) + c2[:, None]
        dmin = jnp.min(scores, axis=0)
        riota = lax.broadcasted_iota(jnp.int32, scores.shape, 0)
        idx = jnp.min(jnp.where(scores == dmin[None, :], riota, NUM_EMB),
                      axis=0).astype(jnp.int32)
        idx_ref[0, b, :] = idx
        part = part + jnp.sum(dmin)

    loss_ref[0, 0] += part


_sc_mesh = plsc.VectorSubcoreMesh(core_axis_name="c", subcore_axis_name="s")


@functools.partial(
    pl.kernel, mesh=_sc_mesh,
    out_type=jax.ShapeDtypeStruct((ROWS, PADW), jnp.float32),
    scratch_types=[
        pltpu.VMEM((BPW,), jnp.int32),
        pltpu.VMEM((BPW, PADW), jnp.float32),
        pltpu.SemaphoreType.DMA,
    ],
)
def _sc_gather(cb_hbm, idx_hbm, out_hbm, idx_v, rows_v, sem):
    wid = lax.axis_index("s") * _NC + lax.axis_index("c")
    base = wid * BPW
    pltpu.sync_copy(idx_hbm.at[pl.ds(base, BPW)], idx_v)
    pltpu.async_copy(cb_hbm.at[idx_v], rows_v, sem).wait()
    pltpu.sync_copy(rows_v, out_hbm.at[pl.ds(base, BPW)])


def kernel(z, codebook):
    zt = jnp.swapaxes(z, 1, 2)
    cbt = codebook.T
    idx, loss_sum, cb_pad = pl.pallas_call(
        _vq_body,
        grid=(NBLK,),
        in_specs=[
            pl.BlockSpec((BB, DIM, NT), lambda i: (i, 0, 0)),
            pl.BlockSpec((DIM, NUM_EMB), lambda i: (0, 0)),
        ],
        out_specs=[
            pl.BlockSpec((1, BB, NT), lambda i: (i, 0, 0)),
            pl.BlockSpec(memory_space=pltpu.SMEM,
                         block_shape=(1, 1), index_map=lambda i: (0, 0)),
            pl.BlockSpec((NUM_EMB, PADW), lambda i: (0, 0)),
        ],
        out_shape=[
            jax.ShapeDtypeStruct((NBLK, BB, NT), jnp.int32),
            jax.ShapeDtypeStruct((1, 1), jnp.float32),
            jax.ShapeDtypeStruct((NUM_EMB, PADW), jnp.float32),
        ],
    )(zt, cbt)
    idx_flat = idx.reshape(ROWS)
    q = _sc_gather(cb_pad, idx_flat)
    quantized = q[:, :DIM].reshape(z.shape)
    indices = idx_flat.reshape(z.shape[0], -1)
    loss = (1.0 + CCOST) * loss_sum[0, 0] / (ROWS * DIM)
    return quantized, indices, loss


# speedup vs baseline: 1.0956x; 1.0058x over previous
import functools

import jax
import jax.numpy as jnp
from jax import lax
from jax.experimental import pallas as pl
from jax.experimental.pallas import tpu as pltpu
from jax.experimental.pallas import tpu_sc as plsc

NUM_EMB = 1024
DIM = 64
CCOST = 0.25
NB, NT = 16, 576
ROWS = NB * NT
BB = 4
NBLK = NB // BB
PADW = 128

_info = plsc.get_sparse_core_info()
_NC, _NS = _info.num_cores, _info.num_subcores
NW = _NC * _NS
BPW = ROWS // NW


def _vq_body(zt_ref, cbt_ref, idx_ref, loss_ref, cbp_ref):
    i = pl.program_id(0)
    cbt = cbt_ref[...]

    @pl.when(i == 0)
    def _init():
        loss_ref[0, 0] = 0.0
        cbp_ref[:, :DIM] = cbt.T
        cbp_ref[:, DIM:] = jnp.zeros((NUM_EMB, PADW - DIM), jnp.float32)

    c2 = jnp.sum(cbt * cbt, axis=0)
    part = jnp.float32(0.0)
    for b in range(BB):
        zt = zt_ref[b]
        z2 = jnp.sum(zt * zt, axis=0)
        mt = jax.lax.dot_general(cbt, zt, (((0,), (0,)), ((), ())),
                                 preferred_element_type=jnp.float32)
        scores = (z2[None, :] - 2.0 * mt) + c2[:, None]
        dmin = jnp.min(scores, axis=0)
        riota = lax.broadcasted_iota(jnp.int32, scores.shape, 0)
        idx = jnp.min(jnp.where(scores == dmin[None, :], riota, NUM_EMB),
                      axis=0).astype(jnp.int32)
        idx_ref[0, b, :] = idx
        part = part + jnp.sum(dmin)

    loss_ref[0, 0] += part


_sc_mesh = plsc.VectorSubcoreMesh(core_axis_name="c", subcore_axis_name="s")


@functools.partial(
    pl.kernel, mesh=_sc_mesh,
    out_type=jax.ShapeDtypeStruct((ROWS, PADW), jnp.float32),
    scratch_types=[
        pltpu.VMEM((BPW,), jnp.int32),
        pltpu.VMEM((BPW, PADW), jnp.float32),
        pltpu.SemaphoreType.DMA,
    ],
)
def _sc_gather(cb_hbm, idx_hbm, out_hbm, idx_v, rows_v, sem):
    wid = lax.axis_index("s") * _NC + lax.axis_index("c")
    base = wid * BPW
    pltpu.sync_copy(idx_hbm.at[pl.ds(base, BPW)], idx_v)
    pltpu.async_copy(cb_hbm.at[idx_v], rows_v, sem).wait()
    pltpu.sync_copy(rows_v, out_hbm.at[pl.ds(base, BPW)])


def kernel(z, codebook):
    zt = jnp.swapaxes(z, 1, 2)
    cbt = codebook.T
    idx, loss_sum, cb_pad = pl.pallas_call(
        _vq_body,
        grid=(NBLK,),
        in_specs=[
            pl.BlockSpec((BB, DIM, NT), lambda i: (i, 0, 0)),
            pl.BlockSpec((DIM, NUM_EMB), lambda i: (0, 0)),
        ],
        out_specs=[
            pl.BlockSpec((1, BB, NT), lambda i: (i, 0, 0)),
            pl.BlockSpec(memory_space=pltpu.SMEM,
                         block_shape=(1, 1), index_map=lambda i: (0, 0)),
            pl.BlockSpec((NUM_EMB, PADW), lambda i: (0, 0)),
        ],
        out_shape=[
            jax.ShapeDtypeStruct((NBLK, BB, NT), jnp.int32),
            jax.ShapeDtypeStruct((1, 1), jnp.float32),
            jax.ShapeDtypeStruct((NUM_EMB, PADW), jnp.float32),
        ],
    )(zt, cbt)
    idx_flat = idx.reshape(ROWS)
    q = _sc_gather(cb_pad, idx_flat)
    quantized = q[:, :DIM].reshape(z.shape)
    indices = idx_flat.reshape(z.shape[0], -1)
    loss = (1.0 + CCOST) * loss_sum[0, 0] / (ROWS * DIM)
    return quantized, indices, loss
